# Initial kernel scaffold; baseline (speedup 1.0000x reference)
import functools

import jax
import jax.numpy as jnp
from jax import lax
from jax.experimental import pallas as pl
from jax.experimental.pallas import tpu as pltpu
from jax.experimental.pallas import tpu_sc as plsc

N = 10000
NP = 10240
E = 320000
DF = 128
DE = 16
H = 128
EEMB = 64
K = 128
CH = 79
NC = 2
NS = 16
NW = NC * NS
EP = NW * CH * K
STRIPE = NP // NS

f32 = jnp.float32


def _mesh():
    return plsc.VectorSubcoreMesh(core_axis_name="c", subcore_axis_name="s")


def _sc_degree(dst3, ones16, zeros16):
    @functools.partial(
        pl.kernel,
        out_type=jax.ShapeDtypeStruct((NC, NP, 16), f32),
        mesh=_mesh(),
        scratch_types=[
            pltpu.VMEM((CH, K), jnp.int32),
            pltpu.VMEM((K, 16), f32),
            pltpu.VMEM_SHARED((NP, 16), f32),
        ],
    )
    def deg_kernel(dst_hbm, ones_hbm, zeros_hbm, out_hbm, dst_v, ones_v, acc_sh):
        c = lax.axis_index("c")
        s = lax.axis_index("s")
        w = c * NS + s
        pltpu.sync_copy(dst_hbm.at[w], dst_v)
        pltpu.sync_copy(ones_hbm, ones_v)
        pltpu.sync_copy(zeros_hbm.at[pl.ds(s * STRIPE, STRIPE)],
                        acc_sh.at[pl.ds(s * STRIPE, STRIPE)])
        plsc.subcore_barrier()

        def chunk(j, carry):
            pltpu.sync_copy(ones_v, acc_sh.at[dst_v.at[j]], add=True)
            return carry

        lax.fori_loop(0, CH, chunk, 0)
        plsc.subcore_barrier()
        pltpu.sync_copy(acc_sh.at[pl.ds(s * STRIPE, STRIPE)],
                        out_hbm.at[c, pl.ds(s * STRIPE, STRIPE)])

    return deg_kernel(dst3, ones16, zeros16)


def _sc_segsum(y, src3, dst3, zerosN):
    @functools.partial(
        pl.kernel,
        out_type=jax.ShapeDtypeStruct((NC, NP, H), f32),
        mesh=_mesh(),
        scratch_types=[
            pltpu.VMEM((CH, K), jnp.int32),
            pltpu.VMEM((CH, K), jnp.int32),
            pltpu.VMEM((K, H), f32),
            pltpu.VMEM_SHARED((NP, H), f32),
            pltpu.SemaphoreType.DMA,
        ],
    )
    def seg_kernel(y_hbm, src_hbm, dst_hbm, zeros_hbm, out_hbm,
                   src_v, dst_v, rows_v, acc_sh, sem):
        c = lax.axis_index("c")
        s = lax.axis_index("s")
        w = c * NS + s
        pltpu.sync_copy(src_hbm.at[w], src_v)
        pltpu.sync_copy(dst_hbm.at[w], dst_v)
        pltpu.sync_copy(zeros_hbm.at[pl.ds(s * STRIPE, STRIPE)],
                        acc_sh.at[pl.ds(s * STRIPE, STRIPE)])
        plsc.subcore_barrier()

        def chunk(j, carry):
            pltpu.async_copy(y_hbm.at[src_v.at[j]], rows_v, sem).wait()
            pltpu.sync_copy(rows_v, acc_sh.at[dst_v.at[j]], add=True)
            return carry

        lax.fori_loop(0, CH, chunk, 0)
        plsc.subcore_barrier()
        pltpu.sync_copy(acc_sh.at[pl.ds(s * STRIPE, STRIPE)],
                        out_hbm.at[c, pl.ds(s * STRIPE, STRIPE)])

    return seg_kernel(y, src3, dst3, zerosN)


def _sc_gather_add(a_arr, b_arr, src3, dst3):
    @functools.partial(
        pl.kernel,
        out_type=jax.ShapeDtypeStruct((EP, H), f32),
        mesh=_mesh(),
        scratch_types=[
            pltpu.VMEM((CH, K), jnp.int32),
            pltpu.VMEM((CH, K), jnp.int32),
            pltpu.VMEM((K, H), f32),
            pltpu.VMEM((K, H), f32),
            pltpu.SemaphoreType.DMA,
            pltpu.SemaphoreType.DMA,
        ],
    )
    def ga_kernel(a_hbm, b_hbm, src_hbm, dst_hbm, out_hbm,
                  src_v, dst_v, bufa, bufb, sema, semb):
        c = lax.axis_index("c")
        s = lax.axis_index("s")
        w = c * NS + s
        pltpu.sync_copy(src_hbm.at[w], src_v)
        pltpu.sync_copy(dst_hbm.at[w], dst_v)

        def chunk(j, carry):
            cpa = pltpu.async_copy(a_hbm.at[src_v.at[j]], bufa, sema)
            cpb = pltpu.async_copy(b_hbm.at[dst_v.at[j]], bufb, semb)
            cpa.wait()
            cpb.wait()

            def vrow(r, carry2):
                for q in range(H // 16):
                    sl = pl.ds(q * 16, 16)
                    bufa[r, sl] = bufa[r, sl] + bufb[r, sl]
                return carry2

            lax.fori_loop(0, K, vrow, 0)
            pltpu.sync_copy(bufa, out_hbm.at[pl.ds((w * CH + j) * K, K)])
            return carry

        lax.fori_loop(0, CH, chunk, 0)

    return ga_kernel(a_arr, b_arr, src3, dst3)


def _full(shape):
    nd = len(shape)
    return pl.BlockSpec(shape, lambda i: (0,) * nd)


def _tc_embed(x_p, degp, ne_W1, ne_b1, ne_W2, ne_b2, g_W1):
    bn = 1024

    def body(x_ref, deg_ref, w1_ref, b1_ref, w2_ref, b2_ref, gw_ref,
             y_ref, dis_ref):
        t = jax.nn.relu(jnp.dot(x_ref[...], w1_ref[...],
                                preferred_element_type=f32) + b1_ref[...])
        ne = jax.nn.relu(jnp.dot(t, w2_ref[...],
                                 preferred_element_type=f32) + b2_ref[...])
        xw = jnp.dot(ne, gw_ref[...], preferred_element_type=f32)
        degf = deg_ref[...]
        deg = degf[0, :, :1] + degf[1, :, :1] + 1.0
        dis = lax.rsqrt(deg)
        y_ref[...] = xw * dis
        dis_ref[...] = dis

    return pl.pallas_call(
        body,
        grid=(NP // bn,),
        in_specs=[
            pl.BlockSpec((bn, DF), lambda i: (i, 0)),
            pl.BlockSpec((NC, bn, 16), lambda i: (0, i, 0)),
            _full((DF, H)),
            _full((H,)),
            _full((H, H)),
            _full((H,)),
            _full((H, H)),
        ],
        out_specs=[
            pl.BlockSpec((bn, H), lambda i: (i, 0)),
            pl.BlockSpec((bn, 1), lambda i: (i, 0)),
        ],
        out_shape=[
            jax.ShapeDtypeStruct((NP, H), f32),
            jax.ShapeDtypeStruct((NP, 1), f32),
        ],
    )(x_p, degp, ne_W1, ne_b1, ne_W2, ne_b2, g_W1)


def _tc_layer(sparts, y, dis, b, W):
    bn = 1024

    def body(s_ref, y_ref, dis_ref, b_ref, w_ref, o_ref):
        sf = s_ref[...]
        dis = dis_ref[...]
        h = jax.nn.relu((sf[0] + sf[1] + y_ref[...]) * dis + b_ref[...])
        o_ref[...] = jnp.dot(h, w_ref[...], preferred_element_type=f32) * dis

    return pl.pallas_call(
        body,
        grid=(NP // bn,),
        in_specs=[
            pl.BlockSpec((NC, bn, H), lambda i: (0, i, 0)),
            pl.BlockSpec((bn, H), lambda i: (i, 0)),
            pl.BlockSpec((bn, 1), lambda i: (i, 0)),
            _full((H,)),
            _full((H, H)),
        ],
        out_specs=pl.BlockSpec((bn, H), lambda i: (i, 0)),
        out_shape=jax.ShapeDtypeStruct((NP, H), f32),
    )(sparts, y, dis, b, W)


def _tc_final(sparts, y, dis, b, W1a, W1b):
    bn = 1024

    def body(s_ref, y_ref, dis_ref, b_ref, wa_ref, wb_ref, a_ref, bo_ref):
        sf = s_ref[...]
        h3 = (sf[0] + sf[1] + y_ref[...]) * dis_ref[...] + b_ref[...]
        a_ref[...] = jnp.dot(h3, wa_ref[...], preferred_element_type=f32)
        bo_ref[...] = jnp.dot(h3, wb_ref[...], preferred_element_type=f32)

    return pl.pallas_call(
        body,
        grid=(NP // bn,),
        in_specs=[
            pl.BlockSpec((NC, bn, H), lambda i: (0, i, 0)),
            pl.BlockSpec((bn, H), lambda i: (i, 0)),
            pl.BlockSpec((bn, 1), lambda i: (i, 0)),
            _full((H,)),
            _full((H, H)),
            _full((H, H)),
        ],
        out_specs=[
            pl.BlockSpec((bn, H), lambda i: (i, 0)),
            pl.BlockSpec((bn, H), lambda i: (i, 0)),
        ],
        out_shape=[
            jax.ShapeDtypeStruct((NP, H), f32),
            jax.ShapeDtypeStruct((NP, H), f32),
        ],
    )(sparts, y, dis, b, W1a, W1b)


def _tc_head(G, ea_p, ee_W1, ee_b1, ee_W2, ee_b2, W1c,
             h_b1, h_W2, h_b2, h_W3, h_b3):
    be = 2048

    def body(g_ref, ea_ref, ew1, eb1, ew2, eb2, w1c, b1, w2, b2, w3, b3,
             o_ref):
        t = jax.nn.relu(jnp.dot(ea_ref[...], ew1[...],
                                preferred_element_type=f32) + eb1[...])
        emb = jax.nn.relu(jnp.dot(t, ew2[...],
                                  preferred_element_type=f32) + eb2[...])
        z1 = jax.nn.relu(g_ref[...] + jnp.dot(emb, w1c[...],
                                              preferred_element_type=f32)
                         + b1[...])
        z2 = jax.nn.relu(jnp.dot(z1, w2[...],
                                 preferred_element_type=f32) + b2[...])
        o_ref[...] = jnp.dot(z2, w3[...], preferred_element_type=f32) + b3[...]

    return pl.pallas_call(
        body,
        grid=(EP // be,),
        in_specs=[
            pl.BlockSpec((be, H), lambda i: (i, 0)),
            pl.BlockSpec((be, DE), lambda i: (i, 0)),
            _full((DE, H)),
            _full((H,)),
            _full((H, EEMB)),
            _full((EEMB,)),
            _full((EEMB, H)),
            _full((H,)),
            _full((H, EEMB)),
            _full((EEMB,)),
            _full((EEMB, 1)),
            _full((1,)),
        ],
        out_specs=pl.BlockSpec((be, 1), lambda i: (i, 0)),
        out_shape=jax.ShapeDtypeStruct((EP, 1), f32),
    )(G, ea_p, ee_W1, ee_b1, ee_W2, ee_b2, W1c, h_b1, h_W2, h_b2, h_W3, h_b3)


def kernel(x, edge_index, edge_attr,
           ne_W1, ne_b1, ne_W2, ne_b2,
           ee_W1, ee_b1, ee_W2, ee_b2,
           g_W1, g_b1, g_W2, g_b2, g_W3, g_b3,
           h_W1, h_b1, h_W2, h_b2, h_W3, h_b3):
    x_p = jnp.pad(x, ((0, NP - N), (0, 0)))
    ea_p = jnp.pad(edge_attr, ((0, EP - E), (0, 0)))
    src3 = jnp.pad(edge_index[0], (0, EP - E)).reshape(NW, CH, K)
    dst3 = jnp.pad(edge_index[1], (0, EP - E),
                   constant_values=N).reshape(NW, CH, K)
    ones16 = jnp.ones((K, 16), f32)
    zeros16 = jnp.zeros((NP, 16), f32)
    zerosN = jnp.zeros((NP, H), f32)

    degp = _sc_degree(dst3, ones16, zeros16)
    y1, dis = _tc_embed(x_p, degp, ne_W1, ne_b1, ne_W2, ne_b2, g_W1)
    s1 = _sc_segsum(y1, src3, dst3, zerosN)
    y2 = _tc_layer(s1, y1, dis, g_b1, g_W2)
    s2 = _sc_segsum(y2, src3, dst3, zerosN)
    y3 = _tc_layer(s2, y2, dis, g_b2, g_W3)
    s3 = _sc_segsum(y3, src3, dst3, zerosN)
    W1a = h_W1[0:H]
    W1b = h_W1[H:2 * H]
    W1c = h_W1[2 * H:]
    a_arr, b_arr = _tc_final(s3, y3, dis, g_b3, W1a, W1b)
    G = _sc_gather_add(a_arr, b_arr, src3, dst3)
    pred = _tc_head(G, ea_p, ee_W1, ee_b1, ee_W2, ee_b2, W1c,
                    h_b1, h_W2, h_b2, h_W3, h_b3)
    return pred[:E, 0]


# speedup vs baseline: 6.7642x; 6.7642x over previous
import functools

import jax
import jax.numpy as jnp
from jax import lax
from jax.experimental import pallas as pl
from jax.experimental.pallas import tpu as pltpu
from jax.experimental.pallas import tpu_sc as plsc

N = 10000
NP = 10240
E = 320000
DF = 128
DE = 16
H = 128
EEMB = 64
K = 128
CH = 79
NC = 2
NS = 16
NW = NC * NS
EP = NW * CH * K
STRIPE = NP // NS

f32 = jnp.float32


def _mesh():
    return plsc.VectorSubcoreMesh(core_axis_name="c", subcore_axis_name="s")


def _sc_degree(dst3, ones_rows, zerosN):
    @functools.partial(
        pl.kernel,
        out_type=jax.ShapeDtypeStruct((NC, NP, H), f32),
        mesh=_mesh(),
        scratch_types=[
            pltpu.VMEM((CH, K), jnp.int32),
            pltpu.VMEM((K, H), f32),
            pltpu.VMEM_SHARED((NP, H), f32),
        ],
    )
    def deg_kernel(dst_hbm, ones_hbm, zeros_hbm, out_hbm, dst_v, ones_v, acc_sh):
        c = lax.axis_index("c")
        s = lax.axis_index("s")
        w = c * NS + s
        pltpu.sync_copy(dst_hbm.at[w], dst_v)
        pltpu.sync_copy(ones_hbm, ones_v)
        pltpu.sync_copy(zeros_hbm.at[pl.ds(s * STRIPE, STRIPE)],
                        acc_sh.at[pl.ds(s * STRIPE, STRIPE)])
        plsc.subcore_barrier()

        def chunk(j, carry):
            pltpu.sync_copy(ones_v, acc_sh.at[dst_v.at[j]], add=True)
            return carry

        lax.fori_loop(0, CH, chunk, 0)
        plsc.subcore_barrier()
        pltpu.sync_copy(acc_sh.at[pl.ds(s * STRIPE, STRIPE)],
                        out_hbm.at[c, pl.ds(s * STRIPE, STRIPE)])

    return deg_kernel(dst3, ones_rows, zerosN)


def _sc_segsum(y, src3, dst3, zerosN):
    @functools.partial(
        pl.kernel,
        out_type=jax.ShapeDtypeStruct((NC, NP, H), f32),
        mesh=_mesh(),
        scratch_types=[
            pltpu.VMEM((CH, K), jnp.int32),
            pltpu.VMEM((CH, K), jnp.int32),
            pltpu.VMEM((K, H), f32),
            pltpu.VMEM_SHARED((NP, H), f32),
            pltpu.SemaphoreType.DMA,
        ],
    )
    def seg_kernel(y_hbm, src_hbm, dst_hbm, zeros_hbm, out_hbm,
                   src_v, dst_v, rows_v, acc_sh, sem):
        c = lax.axis_index("c")
        s = lax.axis_index("s")
        w = c * NS + s
        pltpu.sync_copy(src_hbm.at[w], src_v)
        pltpu.sync_copy(dst_hbm.at[w], dst_v)
        pltpu.sync_copy(zeros_hbm.at[pl.ds(s * STRIPE, STRIPE)],
                        acc_sh.at[pl.ds(s * STRIPE, STRIPE)])
        plsc.subcore_barrier()

        def chunk(j, carry):
            pltpu.async_copy(y_hbm.at[src_v.at[j]], rows_v, sem).wait()
            pltpu.sync_copy(rows_v, acc_sh.at[dst_v.at[j]], add=True)
            return carry

        lax.fori_loop(0, CH, chunk, 0)
        plsc.subcore_barrier()
        pltpu.sync_copy(acc_sh.at[pl.ds(s * STRIPE, STRIPE)],
                        out_hbm.at[c, pl.ds(s * STRIPE, STRIPE)])

    return seg_kernel(y, src3, dst3, zerosN)


def _sc_gather_add(a_arr, b_arr, src3, dst3):
    @functools.partial(
        pl.kernel,
        out_type=jax.ShapeDtypeStruct((EP, H), f32),
        mesh=_mesh(),
        scratch_types=[
            pltpu.VMEM((CH, K), jnp.int32),
            pltpu.VMEM((CH, K), jnp.int32),
            pltpu.VMEM((K, H), f32),
            pltpu.VMEM((K, H), f32),
            pltpu.SemaphoreType.DMA,
            pltpu.SemaphoreType.DMA,
        ],
    )
    def ga_kernel(a_hbm, b_hbm, src_hbm, dst_hbm, out_hbm,
                  src_v, dst_v, bufa, bufb, sema, semb):
        c = lax.axis_index("c")
        s = lax.axis_index("s")
        w = c * NS + s
        pltpu.sync_copy(src_hbm.at[w], src_v)
        pltpu.sync_copy(dst_hbm.at[w], dst_v)

        def chunk(j, carry):
            cpa = pltpu.async_copy(a_hbm.at[src_v.at[j]], bufa, sema)
            cpb = pltpu.async_copy(b_hbm.at[dst_v.at[j]], bufb, semb)
            cpa.wait()
            cpb.wait()

            def vrow(r, carry2):
                for q in range(H // 16):
                    sl = pl.ds(q * 16, 16)
                    bufa[r, sl] = bufa[r, sl] + bufb[r, sl]
                return carry2

            lax.fori_loop(0, K, vrow, 0)
            pltpu.sync_copy(bufa, out_hbm.at[pl.ds((w * CH + j) * K, K)])
            return carry

        lax.fori_loop(0, CH, chunk, 0)

    return ga_kernel(a_arr, b_arr, src3, dst3)


def _full(shape):
    nd = len(shape)
    return pl.BlockSpec(shape, lambda i: (0,) * nd)


def _tc_embed(x_p, degp, ne_W1, ne_b1, ne_W2, ne_b2, g_W1):
    bn = 1024

    def body(x_ref, deg_ref, w1_ref, b1_ref, w2_ref, b2_ref, gw_ref,
             y_ref, dis_ref):
        t = jax.nn.relu(jnp.dot(x_ref[...], w1_ref[...],
                                preferred_element_type=f32) + b1_ref[...])
        ne = jax.nn.relu(jnp.dot(t, w2_ref[...],
                                 preferred_element_type=f32) + b2_ref[...])
        xw = jnp.dot(ne, gw_ref[...], preferred_element_type=f32)
        degf = deg_ref[...]
        deg = degf[0, :, :1] + degf[1, :, :1] + 1.0
        dis = lax.rsqrt(deg)
        y_ref[...] = xw * dis
        dis_ref[...] = dis

    return pl.pallas_call(
        body,
        grid=(NP // bn,),
        in_specs=[
            pl.BlockSpec((bn, DF), lambda i: (i, 0)),
            pl.BlockSpec((NC, bn, H), lambda i: (0, i, 0)),
            _full((DF, H)),
            _full((H,)),
            _full((H, H)),
            _full((H,)),
            _full((H, H)),
        ],
        out_specs=[
            pl.BlockSpec((bn, H), lambda i: (i, 0)),
            pl.BlockSpec((bn, 1), lambda i: (i, 0)),
        ],
        out_shape=[
            jax.ShapeDtypeStruct((NP, H), f32),
            jax.ShapeDtypeStruct((NP, 1), f32),
        ],
    )(x_p, degp, ne_W1, ne_b1, ne_W2, ne_b2, g_W1)


def _tc_layer(sparts, y, dis, b, W):
    bn = 1024

    def body(s_ref, y_ref, dis_ref, b_ref, w_ref, o_ref):
        sf = s_ref[...]
        dis = dis_ref[...]
        h = jax.nn.relu((sf[0] + sf[1] + y_ref[...]) * dis + b_ref[...])
        o_ref[...] = jnp.dot(h, w_ref[...], preferred_element_type=f32) * dis

    return pl.pallas_call(
        body,
        grid=(NP // bn,),
        in_specs=[
            pl.BlockSpec((NC, bn, H), lambda i: (0, i, 0)),
            pl.BlockSpec((bn, H), lambda i: (i, 0)),
            pl.BlockSpec((bn, 1), lambda i: (i, 0)),
            _full((H,)),
            _full((H, H)),
        ],
        out_specs=pl.BlockSpec((bn, H), lambda i: (i, 0)),
        out_shape=jax.ShapeDtypeStruct((NP, H), f32),
    )(sparts, y, dis, b, W)


def _tc_final(sparts, y, dis, b, W1a, W1b):
    bn = 1024

    def body(s_ref, y_ref, dis_ref, b_ref, wa_ref, wb_ref, a_ref, bo_ref):
        sf = s_ref[...]
        h3 = (sf[0] + sf[1] + y_ref[...]) * dis_ref[...] + b_ref[...]
        a_ref[...] = jnp.dot(h3, wa_ref[...], preferred_element_type=f32)
        bo_ref[...] = jnp.dot(h3, wb_ref[...], preferred_element_type=f32)

    return pl.pallas_call(
        body,
        grid=(NP // bn,),
        in_specs=[
            pl.BlockSpec((NC, bn, H), lambda i: (0, i, 0)),
            pl.BlockSpec((bn, H), lambda i: (i, 0)),
            pl.BlockSpec((bn, 1), lambda i: (i, 0)),
            _full((H,)),
            _full((H, H)),
            _full((H, H)),
        ],
        out_specs=[
            pl.BlockSpec((bn, H), lambda i: (i, 0)),
            pl.BlockSpec((bn, H), lambda i: (i, 0)),
        ],
        out_shape=[
            jax.ShapeDtypeStruct((NP, H), f32),
            jax.ShapeDtypeStruct((NP, H), f32),
        ],
    )(sparts, y, dis, b, W1a, W1b)


def _tc_head(G, ea_p, ee_W1, ee_b1, ee_W2, ee_b2, W1c,
             h_b1, h_W2, h_b2, h_W3, h_b3):
    be = 2048

    def body(g_ref, ea_ref, ew1, eb1, ew2, eb2, w1c, b1, w2, b2, w3, b3,
             o_ref):
        t = jax.nn.relu(jnp.dot(ea_ref[...], ew1[...],
                                preferred_element_type=f32) + eb1[...])
        emb = jax.nn.relu(jnp.dot(t, ew2[...],
                                  preferred_element_type=f32) + eb2[...])
        z1 = jax.nn.relu(g_ref[...] + jnp.dot(emb, w1c[...],
                                              preferred_element_type=f32)
                         + b1[...])
        z2 = jax.nn.relu(jnp.dot(z1, w2[...],
                                 preferred_element_type=f32) + b2[...])
        o_ref[...] = jnp.dot(z2, w3[...], preferred_element_type=f32) + b3[...]

    return pl.pallas_call(
        body,
        grid=(EP // be,),
        in_specs=[
            pl.BlockSpec((be, H), lambda i: (i, 0)),
            pl.BlockSpec((be, DE), lambda i: (i, 0)),
            _full((DE, H)),
            _full((H,)),
            _full((H, EEMB)),
            _full((EEMB,)),
            _full((EEMB, H)),
            _full((H,)),
            _full((H, EEMB)),
            _full((EEMB,)),
            _full((EEMB, 1)),
            _full((1,)),
        ],
        out_specs=pl.BlockSpec((be, 1), lambda i: (i, 0)),
        out_shape=jax.ShapeDtypeStruct((EP, 1), f32),
    )(G, ea_p, ee_W1, ee_b1, ee_W2, ee_b2, W1c, h_b1, h_W2, h_b2, h_W3, h_b3)


def kernel(x, edge_index, edge_attr,
           ne_W1, ne_b1, ne_W2, ne_b2,
           ee_W1, ee_b1, ee_W2, ee_b2,
           g_W1, g_b1, g_W2, g_b2, g_W3, g_b3,
           h_W1, h_b1, h_W2, h_b2, h_W3, h_b3):
    x_p = jnp.pad(x, ((0, NP - N), (0, 0)))
    ea_p = jnp.pad(edge_attr, ((0, EP - E), (0, 0)))
    src3 = jnp.pad(edge_index[0], (0, EP - E)).reshape(NW, CH, K)
    dst3 = jnp.pad(edge_index[1], (0, EP - E),
                   constant_values=N).reshape(NW, CH, K)
    ones_rows = jnp.ones((K, H), f32)
    zerosN = jnp.zeros((NP, H), f32)

    degp = _sc_degree(dst3, ones_rows, zerosN)
    y1, dis = _tc_embed(x_p, degp, ne_W1, ne_b1, ne_W2, ne_b2, g_W1)
    s1 = _sc_segsum(y1, src3, dst3, zerosN)
    y2 = _tc_layer(s1, y1, dis, g_b1, g_W2)
    s2 = _sc_segsum(y2, src3, dst3, zerosN)
    y3 = _tc_layer(s2, y2, dis, g_b2, g_W3)
    s3 = _sc_segsum(y3, src3, dst3, zerosN)
    W1a = h_W1[0:H]
    W1b = h_W1[H:2 * H]
    W1c = h_W1[2 * H:]
    a_arr, b_arr = _tc_final(s3, y3, dis, g_b3, W1a, W1b)
    G = _sc_gather_add(a_arr, b_arr, src3, dst3)
    pred = _tc_head(G, ea_p, ee_W1, ee_b1, ee_W2, ee_b2, W1c,
                    h_b1, h_W2, h_b2, h_W3, h_b3)
    return pred[:E, 0]
